# Initial kernel scaffold; baseline (speedup 1.0000x reference)
import jax
import jax.numpy as jnp
from jax.experimental import pallas as pl


def kernel(x, node_att, edge_index, edge_feat, edge_att, W_node, b_node, W_rel, b_rel, W_apply, b_apply):
    raise NotImplementedError("write your pallas kernel here")


# speedup vs baseline: 3.1007x; 3.1007x over previous
import jax
import jax.numpy as jnp
from jax import lax
from jax.experimental import pallas as pl
from jax.experimental.pallas import tpu as pltpu
from jax.experimental.pallas import tpu_sc as plsc

N_NODES = 10000
N_EDGES = 160000
D = 256
RD = 16
O = 256
HALF = 128

BM = 1000
BE = 2000
CHUNK = 128
N_CHUNKS = N_EDGES // CHUNK
N_SUBCORES = 16
ROWS_PER_TILE = 624
TAIL_BASE = ROWS_PER_TILE * N_SUBCORES
TAIL_ROWS = N_NODES - TAIL_BASE
SLAB = 104
SLABS_PER_TILE = ROWS_PER_TILE // SLAB


def _node_fc_body(x_ref, wnt_ref, bn_ref, att_ref, w1t_ref, h_ref, u_ref):
    h = jnp.dot(x_ref[...], wnt_ref[...], preferred_element_type=jnp.float32)
    h = h + bn_ref[...]
    h_ref[...] = h
    u = jnp.dot(att_ref[...] * h, w1t_ref[...],
                preferred_element_type=jnp.float32)
    u_ref[0] = u[:, :HALF]
    u_ref[1] = u[:, HALF:]


def _edge_fc_body(ef_ref, wrt_ref, br_ref, eatt_ref, w2t_ref, y2_ref):
    z = jnp.dot(ef_ref[...], wrt_ref[...], preferred_element_type=jnp.float32)
    z2 = eatt_ref[...] * (z + br_ref[...])
    y2 = jnp.dot(z2, w2t_ref[...], preferred_element_type=jnp.float32)
    y2_ref[0] = y2[:, :HALF]
    y2_ref[1] = y2[:, HALF:]


def _apply_body(agg_ref, h_ref, att_ref, w3t_ref, b_ref, out_ref):
    full = jnp.dot(h_ref[...], w3t_ref[...], preferred_element_type=jnp.float32)
    full = full + b_ref[...]
    att = att_ref[...]
    out_ref[:, :HALF] = att * jnp.maximum(agg_ref[0] + full[:, :HALF], 0.0)
    out_ref[:, HALF:] = att * jnp.maximum(agg_ref[1] + full[:, HALF:], 0.0)


def _sc_gather_scatter(ucat_hbm, y2cat_hbm, srccat_hbm, dst_hbm, zeros128_hbm,
                       out_hbm,
                       src_v, dst_v, rows_v, rows2_v, sem, buf, accum):
    c = lax.axis_index("c")
    s = lax.axis_index("s")
    r0 = s * ROWS_PER_TILE

    pltpu.sync_copy(zeros128_hbm.at[pl.ds(0, SLAB)], buf)
    for k in range(SLABS_PER_TILE):
        pltpu.sync_copy(buf, accum.at[pl.ds(r0 + k * SLAB, SLAB)])

    @pl.when(s == N_SUBCORES - 1)
    def _():
        pltpu.sync_copy(buf.at[pl.ds(0, TAIL_ROWS)],
                        accum.at[pl.ds(TAIL_BASE, TAIL_ROWS)])

    plsc.subcore_barrier()

    lo = (s * N_CHUNKS) // N_SUBCORES
    hi = ((s + 1) * N_CHUNKS) // N_SUBCORES

    def body(j, carry):
        base = j * CHUNK
        pltpu.sync_copy(srccat_hbm.at[pl.ds(c * N_EDGES + base, CHUNK)], src_v)
        pltpu.sync_copy(dst_hbm.at[pl.ds(base, CHUNK)], dst_v)
        pltpu.async_copy(ucat_hbm.at[src_v], rows_v, sem).wait()
        pltpu.sync_copy(y2cat_hbm.at[pl.ds(c * N_EDGES + base, CHUNK)],
                        rows2_v)
        pltpu.sync_copy(rows_v, accum.at[dst_v], add=True)
        pltpu.sync_copy(rows2_v, accum.at[dst_v], add=True)
        return carry

    lax.fori_loop(lo, hi, body, 0)
    plsc.subcore_barrier()

    for k in range(SLABS_PER_TILE):
        rr = r0 + k * SLAB
        pltpu.sync_copy(accum.at[pl.ds(rr, SLAB)], buf)
        pltpu.sync_copy(buf, out_hbm.at[c, pl.ds(rr, SLAB)])

    @pl.when(s == N_SUBCORES - 1)
    def _():
        pltpu.sync_copy(accum.at[pl.ds(TAIL_BASE, TAIL_ROWS)],
                        buf.at[pl.ds(0, TAIL_ROWS)])
        pltpu.sync_copy(buf.at[pl.ds(0, TAIL_ROWS)],
                        out_hbm.at[c, pl.ds(TAIL_BASE, TAIL_ROWS)])


def kernel(x, node_att, edge_index, edge_feat, edge_att,
           W_node, b_node, W_rel, b_rel, W_apply, b_apply):
    n_blocks = N_NODES // BM
    e_blocks = N_EDGES // BE

    att2 = node_att[:, None]
    eatt2 = edge_att[:, None]
    w1t = W_apply[:, :D].T
    w2t = W_apply[:, D:D + RD].T
    w3t = W_apply[:, D + RD:].T

    h, ucat3 = pl.pallas_call(
        _node_fc_body,
        grid=(n_blocks,),
        in_specs=[
            pl.BlockSpec((BM, D), lambda i: (i, 0)),
            pl.BlockSpec((D, D), lambda i: (0, 0)),
            pl.BlockSpec((1, D), lambda i: (0, 0)),
            pl.BlockSpec((BM, 1), lambda i: (i, 0)),
            pl.BlockSpec((D, O), lambda i: (0, 0)),
        ],
        out_specs=[
            pl.BlockSpec((BM, D), lambda i: (i, 0)),
            pl.BlockSpec((2, BM, HALF), lambda i: (0, i, 0)),
        ],
        out_shape=[
            jax.ShapeDtypeStruct((N_NODES, D), jnp.float32),
            jax.ShapeDtypeStruct((2, N_NODES, HALF), jnp.float32),
        ],
    )(x, W_node.T, b_node[None, :], att2, w1t)

    y2cat3 = pl.pallas_call(
        _edge_fc_body,
        grid=(e_blocks,),
        in_specs=[
            pl.BlockSpec((BE, RD), lambda i: (i, 0)),
            pl.BlockSpec((RD, RD), lambda i: (0, 0)),
            pl.BlockSpec((1, RD), lambda i: (0, 0)),
            pl.BlockSpec((BE, 1), lambda i: (i, 0)),
            pl.BlockSpec((RD, O), lambda i: (0, 0)),
        ],
        out_specs=pl.BlockSpec((2, BE, HALF), lambda i: (0, i, 0)),
        out_shape=jax.ShapeDtypeStruct((2, N_EDGES, HALF), jnp.float32),
    )(edge_feat, W_rel.T, b_rel[None, :], eatt2, w2t)

    ucat = ucat3.reshape(2 * N_NODES, HALF)
    y2cat = y2cat3.reshape(2 * N_EDGES, HALF)
    src = edge_index[0]
    dst = edge_index[1]
    src_cat = jnp.concatenate([src, src + N_NODES])
    zeros128 = jnp.zeros((N_NODES, HALF), jnp.float32)

    sc_mesh = plsc.VectorSubcoreMesh(core_axis_name="c", subcore_axis_name="s",
                                     num_cores=2, num_subcores=N_SUBCORES)
    agg = pl.kernel(
        _sc_gather_scatter,
        out_type=jax.ShapeDtypeStruct((2, N_NODES, HALF), jnp.float32),
        mesh=sc_mesh,
        scratch_types=[
            pltpu.VMEM((CHUNK,), jnp.int32),
            pltpu.VMEM((CHUNK,), jnp.int32),
            pltpu.VMEM((CHUNK, HALF), jnp.float32),
            pltpu.VMEM((CHUNK, HALF), jnp.float32),
            pltpu.SemaphoreType.DMA,
            pltpu.VMEM((SLAB, HALF), jnp.float32),
            pltpu.VMEM_SHARED((N_NODES, HALF), jnp.float32),
        ],
    )(ucat, y2cat, src_cat, dst, zeros128)

    out = pl.pallas_call(
        _apply_body,
        grid=(n_blocks,),
        in_specs=[
            pl.BlockSpec((2, BM, HALF), lambda i: (0, i, 0)),
            pl.BlockSpec((BM, D), lambda i: (i, 0)),
            pl.BlockSpec((BM, 1), lambda i: (i, 0)),
            pl.BlockSpec((D, O), lambda i: (0, 0)),
            pl.BlockSpec((1, O), lambda i: (0, 0)),
        ],
        out_specs=pl.BlockSpec((BM, O), lambda i: (i, 0)),
        out_shape=jax.ShapeDtypeStruct((N_NODES, O), jnp.float32),
    )(agg, h, att2, w3t, b_apply[None, :])

    return out


# speedup vs baseline: 4.1337x; 1.3332x over previous
import jax
import jax.numpy as jnp
from jax import lax
from jax.experimental import pallas as pl
from jax.experimental.pallas import tpu as pltpu
from jax.experimental.pallas import tpu_sc as plsc

N_NODES = 10000
N_EDGES = 160000
D = 256
RD = 16
O = 256
HALF = 128

BM = 1000
BE = 2000
CHUNK = 80
N_CHUNKS = N_EDGES // CHUNK
N_SUBCORES = 16
CH_PER_TILE = N_CHUNKS // N_SUBCORES
ROWS_PER_TILE = 624
TAIL_BASE = ROWS_PER_TILE * N_SUBCORES
TAIL_ROWS = N_NODES - TAIL_BASE
SLAB = 48
SLABS_PER_TILE = ROWS_PER_TILE // SLAB


def _node_fc_body(x_ref, wnt_ref, bn_ref, att_ref, w1t_ref, w3t_ref, ba_ref,
                  v_ref, u_ref):
    h = jnp.dot(x_ref[...], wnt_ref[...], preferred_element_type=jnp.float32)
    h = h + bn_ref[...]
    v_ref[...] = jnp.dot(h, w3t_ref[...],
                         preferred_element_type=jnp.float32) + ba_ref[...]
    u = jnp.dot(att_ref[...] * h, w1t_ref[...],
                preferred_element_type=jnp.float32)
    u_ref[0] = u[:, :HALF]
    u_ref[1] = u[:, HALF:]


def _edge_fc_body(ef_ref, wrt_ref, br_ref, eatt_ref, w2t_ref, y2_ref):
    z = jnp.dot(ef_ref[...], wrt_ref[...], preferred_element_type=jnp.float32)
    z2 = eatt_ref[...] * (z + br_ref[...])
    y2 = jnp.dot(z2, w2t_ref[...], preferred_element_type=jnp.float32)
    y2_ref[0] = y2[:, :HALF]
    y2_ref[1] = y2[:, HALF:]


def _apply_body(agg_ref, v_ref, att_ref, out_ref):
    att = att_ref[...]
    v = v_ref[...]
    out_ref[:, :HALF] = att * jnp.maximum(agg_ref[0] + v[:, :HALF], 0.0)
    out_ref[:, HALF:] = att * jnp.maximum(agg_ref[1] + v[:, HALF:], 0.0)


def _sc_gather_scatter(ucat_hbm, y2cat_hbm, srccat_hbm, dst_hbm, zeros128_hbm,
                       out_hbm,
                       src0, src1, dst0, dst1, rows0, rows1, rows20, rows21,
                       isem0, isem1, gsem0, gsem1, lsem0, lsem1,
                       buf, accum):
    c = lax.axis_index("c")
    s = lax.axis_index("s")
    r0 = s * ROWS_PER_TILE

    pltpu.sync_copy(zeros128_hbm.at[pl.ds(0, SLAB)], buf)
    for k in range(SLABS_PER_TILE):
        pltpu.sync_copy(buf, accum.at[pl.ds(r0 + k * SLAB, SLAB)])

    @pl.when(s == N_SUBCORES - 1)
    def _():
        pltpu.sync_copy(buf.at[pl.ds(0, TAIL_ROWS)],
                        accum.at[pl.ds(TAIL_BASE, TAIL_ROWS)])

    plsc.subcore_barrier()

    lo = s * CH_PER_TILE
    hi = lo + CH_PER_TILE

    def issue_idx(j, src_v, dst_v, isem):
        base = j * CHUNK
        pltpu.async_copy(srccat_hbm.at[pl.ds(c * N_EDGES + base, CHUNK)],
                         src_v, isem)
        pltpu.async_copy(dst_hbm.at[pl.ds(base, CHUNK)], dst_v, isem)

    def drain_idx(src_v, dst_v, isem):
        pltpu.make_async_copy(srccat_hbm.at[pl.ds(0, CHUNK)], src_v,
                              isem).wait()
        pltpu.make_async_copy(dst_hbm.at[pl.ds(0, CHUNK)], dst_v,
                              isem).wait()

    def issue_loads(j, src_v, rows_v, rows2_v, gsem, lsem):
        base = j * CHUNK
        pltpu.async_copy(ucat_hbm.at[src_v], rows_v, gsem)
        pltpu.async_copy(y2cat_hbm.at[pl.ds(c * N_EDGES + base, CHUNK)],
                         rows2_v, lsem)

    def drain_loads(rows_v, rows2_v, gsem, lsem):
        pltpu.make_async_copy(ucat_hbm.at[pl.ds(0, CHUNK)], rows_v,
                              gsem).wait()
        pltpu.make_async_copy(y2cat_hbm.at[pl.ds(0, CHUNK)], rows2_v,
                              lsem).wait()

    issue_idx(lo, src0, dst0, isem0)
    drain_idx(src0, dst0, isem0)
    issue_loads(lo, src0, rows0, rows20, gsem0, lsem0)
    issue_idx(lo + 1, src1, dst1, isem1)

    def body(j, carry):
        je = j + 1

        @pl.when(lax.rem(j - lo, 2) == 0)
        def _():
            drain_loads(rows0, rows20, gsem0, lsem0)

            @pl.when(je < hi)
            def _():
                drain_idx(src1, dst1, isem1)
                issue_loads(je, src1, rows1, rows21, gsem1, lsem1)

            pltpu.sync_copy(rows0, accum.at[dst0], add=True)
            pltpu.sync_copy(rows20, accum.at[dst0], add=True)

            @pl.when(j + 2 < hi)
            def _():
                issue_idx(j + 2, src0, dst0, isem0)

        @pl.when(lax.rem(j - lo, 2) == 1)
        def _():
            drain_loads(rows1, rows21, gsem1, lsem1)

            @pl.when(je < hi)
            def _():
                drain_idx(src0, dst0, isem0)
                issue_loads(je, src0, rows0, rows20, gsem0, lsem0)

            pltpu.sync_copy(rows1, accum.at[dst1], add=True)
            pltpu.sync_copy(rows21, accum.at[dst1], add=True)

            @pl.when(j + 2 < hi)
            def _():
                issue_idx(j + 2, src1, dst1, isem1)

        return carry

    lax.fori_loop(lo, hi, body, 0)
    plsc.subcore_barrier()

    for k in range(SLABS_PER_TILE):
        rr = r0 + k * SLAB
        pltpu.sync_copy(accum.at[pl.ds(rr, SLAB)], buf)
        pltpu.sync_copy(buf, out_hbm.at[c, pl.ds(rr, SLAB)])

    @pl.when(s == N_SUBCORES - 1)
    def _():
        pltpu.sync_copy(accum.at[pl.ds(TAIL_BASE, TAIL_ROWS)],
                        buf.at[pl.ds(0, TAIL_ROWS)])
        pltpu.sync_copy(buf.at[pl.ds(0, TAIL_ROWS)],
                        out_hbm.at[c, pl.ds(TAIL_BASE, TAIL_ROWS)])


def kernel(x, node_att, edge_index, edge_feat, edge_att,
           W_node, b_node, W_rel, b_rel, W_apply, b_apply):
    n_blocks = N_NODES // BM
    e_blocks = N_EDGES // BE

    att2 = node_att[:, None]
    eatt2 = edge_att[:, None]
    w1t = W_apply[:, :D].T
    w2t = W_apply[:, D:D + RD].T
    w3t = W_apply[:, D + RD:].T

    v, ucat3 = pl.pallas_call(
        _node_fc_body,
        grid=(n_blocks,),
        in_specs=[
            pl.BlockSpec((BM, D), lambda i: (i, 0)),
            pl.BlockSpec((D, D), lambda i: (0, 0)),
            pl.BlockSpec((1, D), lambda i: (0, 0)),
            pl.BlockSpec((BM, 1), lambda i: (i, 0)),
            pl.BlockSpec((D, O), lambda i: (0, 0)),
            pl.BlockSpec((D, O), lambda i: (0, 0)),
            pl.BlockSpec((1, O), lambda i: (0, 0)),
        ],
        out_specs=[
            pl.BlockSpec((BM, O), lambda i: (i, 0)),
            pl.BlockSpec((2, BM, HALF), lambda i: (0, i, 0)),
        ],
        out_shape=[
            jax.ShapeDtypeStruct((N_NODES, O), jnp.float32),
            jax.ShapeDtypeStruct((2, N_NODES, HALF), jnp.float32),
        ],
    )(x, W_node.T, b_node[None, :], att2, w1t, w3t, b_apply[None, :])

    y2cat3 = pl.pallas_call(
        _edge_fc_body,
        grid=(e_blocks,),
        in_specs=[
            pl.BlockSpec((BE, RD), lambda i: (i, 0)),
            pl.BlockSpec((RD, RD), lambda i: (0, 0)),
            pl.BlockSpec((1, RD), lambda i: (0, 0)),
            pl.BlockSpec((BE, 1), lambda i: (i, 0)),
            pl.BlockSpec((RD, O), lambda i: (0, 0)),
        ],
        out_specs=pl.BlockSpec((2, BE, HALF), lambda i: (0, i, 0)),
        out_shape=jax.ShapeDtypeStruct((2, N_EDGES, HALF), jnp.float32),
    )(edge_feat, W_rel.T, b_rel[None, :], eatt2, w2t)

    ucat = ucat3.reshape(2 * N_NODES, HALF)
    y2cat = y2cat3.reshape(2 * N_EDGES, HALF)
    src = edge_index[0]
    dst = edge_index[1]
    src_cat = jnp.concatenate([src, src + N_NODES])
    zeros128 = jnp.zeros((N_NODES, HALF), jnp.float32)

    sc_mesh = plsc.VectorSubcoreMesh(core_axis_name="c", subcore_axis_name="s",
                                     num_cores=2, num_subcores=N_SUBCORES)
    agg = pl.kernel(
        _sc_gather_scatter,
        out_type=jax.ShapeDtypeStruct((2, N_NODES, HALF), jnp.float32),
        mesh=sc_mesh,
        scratch_types=[
            pltpu.VMEM((CHUNK,), jnp.int32),
            pltpu.VMEM((CHUNK,), jnp.int32),
            pltpu.VMEM((CHUNK,), jnp.int32),
            pltpu.VMEM((CHUNK,), jnp.int32),
            pltpu.VMEM((CHUNK, HALF), jnp.float32),
            pltpu.VMEM((CHUNK, HALF), jnp.float32),
            pltpu.VMEM((CHUNK, HALF), jnp.float32),
            pltpu.VMEM((CHUNK, HALF), jnp.float32),
            pltpu.SemaphoreType.DMA,
            pltpu.SemaphoreType.DMA,
            pltpu.SemaphoreType.DMA,
            pltpu.SemaphoreType.DMA,
            pltpu.SemaphoreType.DMA,
            pltpu.SemaphoreType.DMA,
            pltpu.VMEM((SLAB, HALF), jnp.float32),
            pltpu.VMEM_SHARED((N_NODES, HALF), jnp.float32),
        ],
    )(ucat, y2cat, src_cat, dst, zeros128)

    out = pl.pallas_call(
        _apply_body,
        grid=(n_blocks,),
        in_specs=[
            pl.BlockSpec((2, BM, HALF), lambda i: (0, i, 0)),
            pl.BlockSpec((BM, O), lambda i: (i, 0)),
            pl.BlockSpec((BM, 1), lambda i: (i, 0)),
        ],
        out_specs=pl.BlockSpec((BM, O), lambda i: (i, 0)),
        out_shape=jax.ShapeDtypeStruct((N_NODES, O), jnp.float32),
    )(agg, v, att2)

    return out


# speedup vs baseline: 4.5231x; 1.0942x over previous
import jax
import jax.numpy as jnp
from jax import lax
from jax.experimental import pallas as pl
from jax.experimental.pallas import tpu as pltpu
from jax.experimental.pallas import tpu_sc as plsc

N_NODES = 10000
N_EDGES = 160000
D = 256
RD = 16
O = 256
HALF = 128

BM = 1000
BE = 2000
CHUNK = 80
N_CHUNKS = N_EDGES // CHUNK
N_SUBCORES = 16
CH_PER_TILE = N_CHUNKS // N_SUBCORES
ROWS_PER_TILE = 624
TAIL_BASE = ROWS_PER_TILE * N_SUBCORES
TAIL_ROWS = N_NODES - TAIL_BASE
SLAB = 48
SLABS_PER_TILE = ROWS_PER_TILE // SLAB


def _node_fc_body(x_ref, wnt_ref, bn_ref, att_ref, w1t_ref, w3t_ref, ba_ref,
                  v_ref, u_ref):
    h = jnp.dot(x_ref[...], wnt_ref[...], preferred_element_type=jnp.float32)
    h = h + bn_ref[...]
    v_ref[...] = jnp.dot(h, w3t_ref[...],
                         preferred_element_type=jnp.float32) + ba_ref[...]
    u = jnp.dot(att_ref[...] * h, w1t_ref[...],
                preferred_element_type=jnp.float32)
    u_ref[0] = u[:, :HALF]
    u_ref[1] = u[:, HALF:]


def _edge_fc_body(ef_ref, wrt_ref, br_ref, eatt_ref, w2t_ref, y2_ref):
    z = jnp.dot(ef_ref[...], wrt_ref[...], preferred_element_type=jnp.float32)
    z2 = eatt_ref[...] * (z + br_ref[...])
    y2 = jnp.dot(z2, w2t_ref[...], preferred_element_type=jnp.float32)
    y2_ref[0] = y2[:, :HALF]
    y2_ref[1] = y2[:, HALF:]


def _apply_body(agg_ref, v_ref, att_ref, out_ref):
    att = att_ref[...]
    v = v_ref[...]
    out_ref[:, :HALF] = att * jnp.maximum(agg_ref[0] + v[:, :HALF], 0.0)
    out_ref[:, HALF:] = att * jnp.maximum(agg_ref[1] + v[:, HALF:], 0.0)


def _sc_u_scatter(ucat_hbm, src_hbm, dst_hbm, zeros_hbm, out_hbm,
                  src0, src1, dst0, dst1, rows0, rows1,
                  isem0, isem1, gsem0, gsem1, buf, accum):
    c = lax.axis_index("c")
    s = lax.axis_index("s")
    r0 = s * ROWS_PER_TILE

    pltpu.sync_copy(zeros_hbm, buf)
    for k in range(SLABS_PER_TILE):
        pltpu.sync_copy(buf, accum.at[pl.ds(r0 + k * SLAB, SLAB)])

    @pl.when(s == N_SUBCORES - 1)
    def _():
        pltpu.sync_copy(buf.at[pl.ds(0, TAIL_ROWS)],
                        accum.at[pl.ds(TAIL_BASE, TAIL_ROWS)])

    plsc.subcore_barrier()

    lo = s * CH_PER_TILE
    hi = lo + CH_PER_TILE

    def issue_idx(j, src_v, dst_v, isem):
        base = j * CHUNK
        pltpu.async_copy(src_hbm.at[pl.ds(base, CHUNK)], src_v, isem)
        pltpu.async_copy(dst_hbm.at[pl.ds(base, CHUNK)], dst_v, isem)

    def drain_idx(src_v, dst_v, isem):
        pltpu.make_async_copy(src_hbm.at[pl.ds(0, CHUNK)], src_v,
                              isem).wait()
        pltpu.make_async_copy(dst_hbm.at[pl.ds(0, CHUNK)], dst_v,
                              isem).wait()

    def issue_gather(src_v, rows_v, gsem):
        pltpu.async_copy(ucat_hbm.at[c].at[src_v], rows_v, gsem)

    def drain_gather(rows_v, gsem):
        pltpu.make_async_copy(ucat_hbm.at[c, pl.ds(0, CHUNK)], rows_v,
                              gsem).wait()

    issue_idx(lo, src0, dst0, isem0)
    drain_idx(src0, dst0, isem0)
    issue_gather(src0, rows0, gsem0)
    issue_idx(lo + 1, src1, dst1, isem1)

    def body(j, carry):
        je = j + 1

        @pl.when(lax.rem(j - lo, 2) == 0)
        def _():
            drain_gather(rows0, gsem0)

            @pl.when(je < hi)
            def _():
                drain_idx(src1, dst1, isem1)
                issue_gather(src1, rows1, gsem1)

            pltpu.sync_copy(rows0, accum.at[dst0], add=True)

            @pl.when(j + 2 < hi)
            def _():
                issue_idx(j + 2, src0, dst0, isem0)

        @pl.when(lax.rem(j - lo, 2) == 1)
        def _():
            drain_gather(rows1, gsem1)

            @pl.when(je < hi)
            def _():
                drain_idx(src0, dst0, isem0)
                issue_gather(src0, rows0, gsem0)

            pltpu.sync_copy(rows1, accum.at[dst1], add=True)

            @pl.when(j + 2 < hi)
            def _():
                issue_idx(j + 2, src1, dst1, isem1)

        return carry

    lax.fori_loop(lo, hi, body, 0)
    plsc.subcore_barrier()

    for k in range(SLABS_PER_TILE):
        rr = r0 + k * SLAB
        pltpu.sync_copy(accum.at[pl.ds(rr, SLAB)], buf)
        pltpu.sync_copy(buf, out_hbm.at[c, pl.ds(rr, SLAB)])

    @pl.when(s == N_SUBCORES - 1)
    def _():
        pltpu.sync_copy(accum.at[pl.ds(TAIL_BASE, TAIL_ROWS)],
                        buf.at[pl.ds(0, TAIL_ROWS)])
        pltpu.sync_copy(buf.at[pl.ds(0, TAIL_ROWS)],
                        out_hbm.at[c, pl.ds(TAIL_BASE, TAIL_ROWS)])


def _sc_y2_scatter(y2cat_hbm, dst_hbm, aggu_hbm, out_hbm,
                   dst0, dst1, rows0, rows1,
                   isem0, isem1, lsem0, lsem1, buf, accum):
    c = lax.axis_index("c")
    s = lax.axis_index("s")
    r0 = s * ROWS_PER_TILE

    for k in range(SLABS_PER_TILE):
        rr = r0 + k * SLAB
        pltpu.sync_copy(aggu_hbm.at[c, pl.ds(rr, SLAB)], buf)
        pltpu.sync_copy(buf, accum.at[pl.ds(rr, SLAB)])

    @pl.when(s == N_SUBCORES - 1)
    def _():
        pltpu.sync_copy(aggu_hbm.at[c, pl.ds(TAIL_BASE, TAIL_ROWS)],
                        buf.at[pl.ds(0, TAIL_ROWS)])
        pltpu.sync_copy(buf.at[pl.ds(0, TAIL_ROWS)],
                        accum.at[pl.ds(TAIL_BASE, TAIL_ROWS)])

    plsc.subcore_barrier()

    lo = s * CH_PER_TILE
    hi = lo + CH_PER_TILE

    def issue(j, dst_v, rows_v, isem, lsem):
        base = j * CHUNK
        pltpu.async_copy(dst_hbm.at[pl.ds(base, CHUNK)], dst_v, isem)
        pltpu.async_copy(y2cat_hbm.at[c, pl.ds(base, CHUNK)], rows_v, lsem)

    def drain(dst_v, rows_v, isem, lsem):
        pltpu.make_async_copy(dst_hbm.at[pl.ds(0, CHUNK)], dst_v,
                              isem).wait()
        pltpu.make_async_copy(y2cat_hbm.at[c, pl.ds(0, CHUNK)], rows_v,
                              lsem).wait()

    issue(lo, dst0, rows0, isem0, lsem0)

    def body(j, carry):
        je = j + 1

        @pl.when(lax.rem(j - lo, 2) == 0)
        def _():
            drain(dst0, rows0, isem0, lsem0)

            @pl.when(je < hi)
            def _():
                issue(je, dst1, rows1, isem1, lsem1)

            pltpu.sync_copy(rows0, accum.at[dst0], add=True)

        @pl.when(lax.rem(j - lo, 2) == 1)
        def _():
            drain(dst1, rows1, isem1, lsem1)

            @pl.when(je < hi)
            def _():
                issue(je, dst0, rows0, isem0, lsem0)

            pltpu.sync_copy(rows1, accum.at[dst1], add=True)

        return carry

    lax.fori_loop(lo, hi, body, 0)
    plsc.subcore_barrier()

    for k in range(SLABS_PER_TILE):
        rr = r0 + k * SLAB
        pltpu.sync_copy(accum.at[pl.ds(rr, SLAB)], buf)
        pltpu.sync_copy(buf, out_hbm.at[c, pl.ds(rr, SLAB)])

    @pl.when(s == N_SUBCORES - 1)
    def _():
        pltpu.sync_copy(accum.at[pl.ds(TAIL_BASE, TAIL_ROWS)],
                        buf.at[pl.ds(0, TAIL_ROWS)])
        pltpu.sync_copy(buf.at[pl.ds(0, TAIL_ROWS)],
                        out_hbm.at[c, pl.ds(TAIL_BASE, TAIL_ROWS)])


def kernel(x, node_att, edge_index, edge_feat, edge_att,
           W_node, b_node, W_rel, b_rel, W_apply, b_apply):
    n_blocks = N_NODES // BM
    e_blocks = N_EDGES // BE

    att2 = node_att[:, None]
    eatt2 = edge_att[:, None]
    w1t = W_apply[:, :D].T
    w2t = W_apply[:, D:D + RD].T
    w3t = W_apply[:, D + RD:].T

    v, ucat3 = pl.pallas_call(
        _node_fc_body,
        grid=(n_blocks,),
        in_specs=[
            pl.BlockSpec((BM, D), lambda i: (i, 0)),
            pl.BlockSpec((D, D), lambda i: (0, 0)),
            pl.BlockSpec((1, D), lambda i: (0, 0)),
            pl.BlockSpec((BM, 1), lambda i: (i, 0)),
            pl.BlockSpec((D, O), lambda i: (0, 0)),
            pl.BlockSpec((D, O), lambda i: (0, 0)),
            pl.BlockSpec((1, O), lambda i: (0, 0)),
        ],
        out_specs=[
            pl.BlockSpec((BM, O), lambda i: (i, 0)),
            pl.BlockSpec((2, BM, HALF), lambda i: (0, i, 0)),
        ],
        out_shape=[
            jax.ShapeDtypeStruct((N_NODES, O), jnp.float32),
            jax.ShapeDtypeStruct((2, N_NODES, HALF), jnp.float32),
        ],
    )(x, W_node.T, b_node[None, :], att2, w1t, w3t, b_apply[None, :])

    y2cat3 = pl.pallas_call(
        _edge_fc_body,
        grid=(e_blocks,),
        in_specs=[
            pl.BlockSpec((BE, RD), lambda i: (i, 0)),
            pl.BlockSpec((RD, RD), lambda i: (0, 0)),
            pl.BlockSpec((1, RD), lambda i: (0, 0)),
            pl.BlockSpec((BE, 1), lambda i: (i, 0)),
            pl.BlockSpec((RD, O), lambda i: (0, 0)),
        ],
        out_specs=pl.BlockSpec((2, BE, HALF), lambda i: (0, i, 0)),
        out_shape=jax.ShapeDtypeStruct((2, N_EDGES, HALF), jnp.float32),
    )(edge_feat, W_rel.T, b_rel[None, :], eatt2, w2t)

    src = edge_index[0]
    dst = edge_index[1]
    zeros128 = jnp.zeros((SLAB, HALF), jnp.float32)

    sc_mesh = plsc.VectorSubcoreMesh(core_axis_name="c", subcore_axis_name="s",
                                     num_cores=2, num_subcores=N_SUBCORES)
    agg_u = pl.kernel(
        _sc_u_scatter,
        out_type=jax.ShapeDtypeStruct((2, N_NODES, HALF), jnp.float32),
        mesh=sc_mesh,
        scratch_types=[
            pltpu.VMEM((CHUNK,), jnp.int32),
            pltpu.VMEM((CHUNK,), jnp.int32),
            pltpu.VMEM((CHUNK,), jnp.int32),
            pltpu.VMEM((CHUNK,), jnp.int32),
            pltpu.VMEM((CHUNK, HALF), jnp.float32),
            pltpu.VMEM((CHUNK, HALF), jnp.float32),
            pltpu.SemaphoreType.DMA,
            pltpu.SemaphoreType.DMA,
            pltpu.SemaphoreType.DMA,
            pltpu.SemaphoreType.DMA,
            pltpu.VMEM((SLAB, HALF), jnp.float32),
            pltpu.VMEM_SHARED((N_NODES, HALF), jnp.float32),
        ],
    )(ucat3, src, dst, zeros128)

    agg = pl.kernel(
        _sc_y2_scatter,
        out_type=jax.ShapeDtypeStruct((2, N_NODES, HALF), jnp.float32),
        mesh=sc_mesh,
        scratch_types=[
            pltpu.VMEM((CHUNK,), jnp.int32),
            pltpu.VMEM((CHUNK,), jnp.int32),
            pltpu.VMEM((CHUNK, HALF), jnp.float32),
            pltpu.VMEM((CHUNK, HALF), jnp.float32),
            pltpu.SemaphoreType.DMA,
            pltpu.SemaphoreType.DMA,
            pltpu.SemaphoreType.DMA,
            pltpu.SemaphoreType.DMA,
            pltpu.VMEM((SLAB, HALF), jnp.float32),
            pltpu.VMEM_SHARED((N_NODES, HALF), jnp.float32),
        ],
    )(y2cat3, dst, agg_u)

    out = pl.pallas_call(
        _apply_body,
        grid=(n_blocks,),
        in_specs=[
            pl.BlockSpec((2, BM, HALF), lambda i: (0, i, 0)),
            pl.BlockSpec((BM, O), lambda i: (i, 0)),
            pl.BlockSpec((BM, 1), lambda i: (i, 0)),
        ],
        out_specs=pl.BlockSpec((BM, O), lambda i: (i, 0)),
        out_shape=jax.ShapeDtypeStruct((N_NODES, O), jnp.float32),
    )(agg, v, att2)

    return out


# speedup vs baseline: 4.7924x; 1.0595x over previous
import jax
import jax.numpy as jnp
from jax import lax
from jax.experimental import pallas as pl
from jax.experimental.pallas import tpu as pltpu
from jax.experimental.pallas import tpu_sc as plsc

N_NODES = 10000
N_EDGES = 160000
D = 256
RD = 16
O = 256
HALF = 128

BM = 1000
BE = 2000
CHUNK = 128
N_CHUNKS = N_EDGES // CHUNK
N_SUBCORES = 16
ROWS_PER_TILE = 624
TAIL_BASE = ROWS_PER_TILE * N_SUBCORES
TAIL_ROWS = N_NODES - TAIL_BASE
SLAB = 48
SLABS_PER_TILE = ROWS_PER_TILE // SLAB


def _node_fc_body(x_ref, wnt_ref, bn_ref, att_ref, w1t_ref, w3t_ref, ba_ref,
                  v_ref, u_ref):
    h = jnp.dot(x_ref[...], wnt_ref[...], preferred_element_type=jnp.float32)
    h = h + bn_ref[...]
    v_ref[...] = jnp.dot(h, w3t_ref[...],
                         preferred_element_type=jnp.float32) + ba_ref[...]
    u = jnp.dot(att_ref[...] * h, w1t_ref[...],
                preferred_element_type=jnp.float32)
    u_ref[0] = u[:, :HALF]
    u_ref[1] = u[:, HALF:]


def _edge_fc_body(ef_ref, wrt_ref, br_ref, eatt_ref, w2t_ref, y2_ref):
    z = jnp.dot(ef_ref[...], wrt_ref[...], preferred_element_type=jnp.float32)
    z2 = eatt_ref[...] * (z + br_ref[...])
    y2 = jnp.dot(z2, w2t_ref[...], preferred_element_type=jnp.float32)
    y2_ref[0] = y2[:, :HALF]
    y2_ref[1] = y2[:, HALF:]


def _apply_body(agg_ref, v_ref, att_ref, out_ref):
    att = att_ref[...]
    v = v_ref[...]
    out_ref[:, :HALF] = att * jnp.maximum(agg_ref[0] + v[:, :HALF], 0.0)
    out_ref[:, HALF:] = att * jnp.maximum(agg_ref[1] + v[:, HALF:], 0.0)


def _sc_u_scatter(ucat_hbm, src_hbm, dst_hbm, zeros_hbm, out_hbm,
                  src0, src1, dst0, dst1, rows0, rows1,
                  isem0, isem1, gsem0, gsem1, buf, accum):
    c = lax.axis_index("c")
    s = lax.axis_index("s")
    r0 = s * ROWS_PER_TILE

    pltpu.sync_copy(zeros_hbm, buf)
    for k in range(SLABS_PER_TILE):
        pltpu.sync_copy(buf, accum.at[pl.ds(r0 + k * SLAB, SLAB)])

    @pl.when(s == N_SUBCORES - 1)
    def _():
        pltpu.sync_copy(buf.at[pl.ds(0, TAIL_ROWS)],
                        accum.at[pl.ds(TAIL_BASE, TAIL_ROWS)])

    plsc.subcore_barrier()

    lo = (s * N_CHUNKS) // N_SUBCORES
    hi = ((s + 1) * N_CHUNKS) // N_SUBCORES

    def issue_idx(j, src_v, dst_v, isem):
        base = j * CHUNK
        pltpu.async_copy(src_hbm.at[pl.ds(base, CHUNK)], src_v, isem)
        pltpu.async_copy(dst_hbm.at[pl.ds(base, CHUNK)], dst_v, isem)

    def drain_idx(src_v, dst_v, isem):
        pltpu.make_async_copy(src_hbm.at[pl.ds(0, CHUNK)], src_v,
                              isem).wait()
        pltpu.make_async_copy(dst_hbm.at[pl.ds(0, CHUNK)], dst_v,
                              isem).wait()

    def issue_gather(src_v, rows_v, gsem):
        pltpu.async_copy(ucat_hbm.at[c].at[src_v], rows_v, gsem)

    def drain_gather(rows_v, gsem):
        pltpu.make_async_copy(ucat_hbm.at[c, pl.ds(0, CHUNK)], rows_v,
                              gsem).wait()

    issue_idx(lo, src0, dst0, isem0)
    drain_idx(src0, dst0, isem0)
    issue_gather(src0, rows0, gsem0)
    issue_idx(lo + 1, src1, dst1, isem1)

    def body(j, carry):
        je = j + 1

        @pl.when(lax.rem(j - lo, 2) == 0)
        def _():
            drain_gather(rows0, gsem0)

            @pl.when(je < hi)
            def _():
                drain_idx(src1, dst1, isem1)
                issue_gather(src1, rows1, gsem1)

            pltpu.sync_copy(rows0, accum.at[dst0], add=True)

            @pl.when(j + 2 < hi)
            def _():
                issue_idx(j + 2, src0, dst0, isem0)

        @pl.when(lax.rem(j - lo, 2) == 1)
        def _():
            drain_gather(rows1, gsem1)

            @pl.when(je < hi)
            def _():
                drain_idx(src0, dst0, isem0)
                issue_gather(src0, rows0, gsem0)

            pltpu.sync_copy(rows1, accum.at[dst1], add=True)

            @pl.when(j + 2 < hi)
            def _():
                issue_idx(j + 2, src1, dst1, isem1)

        return carry

    lax.fori_loop(lo, hi, body, 0)
    plsc.subcore_barrier()

    for k in range(SLABS_PER_TILE):
        rr = r0 + k * SLAB
        pltpu.sync_copy(accum.at[pl.ds(rr, SLAB)], buf)
        pltpu.sync_copy(buf, out_hbm.at[c, pl.ds(rr, SLAB)])

    @pl.when(s == N_SUBCORES - 1)
    def _():
        pltpu.sync_copy(accum.at[pl.ds(TAIL_BASE, TAIL_ROWS)],
                        buf.at[pl.ds(0, TAIL_ROWS)])
        pltpu.sync_copy(buf.at[pl.ds(0, TAIL_ROWS)],
                        out_hbm.at[c, pl.ds(TAIL_BASE, TAIL_ROWS)])


def _sc_y2_scatter(y2cat_hbm, dst_hbm, aggu_hbm, out_hbm,
                   dst0, dst1, rows0, rows1,
                   isem0, isem1, lsem0, lsem1, buf, accum):
    c = lax.axis_index("c")
    s = lax.axis_index("s")
    r0 = s * ROWS_PER_TILE

    for k in range(SLABS_PER_TILE):
        rr = r0 + k * SLAB
        pltpu.sync_copy(aggu_hbm.at[c, pl.ds(rr, SLAB)], buf)
        pltpu.sync_copy(buf, accum.at[pl.ds(rr, SLAB)])

    @pl.when(s == N_SUBCORES - 1)
    def _():
        pltpu.sync_copy(aggu_hbm.at[c, pl.ds(TAIL_BASE, TAIL_ROWS)],
                        buf.at[pl.ds(0, TAIL_ROWS)])
        pltpu.sync_copy(buf.at[pl.ds(0, TAIL_ROWS)],
                        accum.at[pl.ds(TAIL_BASE, TAIL_ROWS)])

    plsc.subcore_barrier()

    lo = (s * N_CHUNKS) // N_SUBCORES
    hi = ((s + 1) * N_CHUNKS) // N_SUBCORES

    def issue(j, dst_v, rows_v, isem, lsem):
        base = j * CHUNK
        pltpu.async_copy(dst_hbm.at[pl.ds(base, CHUNK)], dst_v, isem)
        pltpu.async_copy(y2cat_hbm.at[c, pl.ds(base, CHUNK)], rows_v, lsem)

    def drain(dst_v, rows_v, isem, lsem):
        pltpu.make_async_copy(dst_hbm.at[pl.ds(0, CHUNK)], dst_v,
                              isem).wait()
        pltpu.make_async_copy(y2cat_hbm.at[c, pl.ds(0, CHUNK)], rows_v,
                              lsem).wait()

    issue(lo, dst0, rows0, isem0, lsem0)

    def body(j, carry):
        je = j + 1

        @pl.when(lax.rem(j - lo, 2) == 0)
        def _():
            drain(dst0, rows0, isem0, lsem0)

            @pl.when(je < hi)
            def _():
                issue(je, dst1, rows1, isem1, lsem1)

            pltpu.sync_copy(rows0, accum.at[dst0], add=True)

        @pl.when(lax.rem(j - lo, 2) == 1)
        def _():
            drain(dst1, rows1, isem1, lsem1)

            @pl.when(je < hi)
            def _():
                issue(je, dst0, rows0, isem0, lsem0)

            pltpu.sync_copy(rows1, accum.at[dst1], add=True)

        return carry

    lax.fori_loop(lo, hi, body, 0)
    plsc.subcore_barrier()

    for k in range(SLABS_PER_TILE):
        rr = r0 + k * SLAB
        pltpu.sync_copy(accum.at[pl.ds(rr, SLAB)], buf)
        pltpu.sync_copy(buf, out_hbm.at[c, pl.ds(rr, SLAB)])

    @pl.when(s == N_SUBCORES - 1)
    def _():
        pltpu.sync_copy(accum.at[pl.ds(TAIL_BASE, TAIL_ROWS)],
                        buf.at[pl.ds(0, TAIL_ROWS)])
        pltpu.sync_copy(buf.at[pl.ds(0, TAIL_ROWS)],
                        out_hbm.at[c, pl.ds(TAIL_BASE, TAIL_ROWS)])


def kernel(x, node_att, edge_index, edge_feat, edge_att,
           W_node, b_node, W_rel, b_rel, W_apply, b_apply):
    n_blocks = N_NODES // BM
    e_blocks = N_EDGES // BE

    att2 = node_att[:, None]
    eatt2 = edge_att[:, None]
    w1t = W_apply[:, :D].T
    w2t = W_apply[:, D:D + RD].T
    w3t = W_apply[:, D + RD:].T

    v, ucat3 = pl.pallas_call(
        _node_fc_body,
        grid=(n_blocks,),
        in_specs=[
            pl.BlockSpec((BM, D), lambda i: (i, 0)),
            pl.BlockSpec((D, D), lambda i: (0, 0)),
            pl.BlockSpec((1, D), lambda i: (0, 0)),
            pl.BlockSpec((BM, 1), lambda i: (i, 0)),
            pl.BlockSpec((D, O), lambda i: (0, 0)),
            pl.BlockSpec((D, O), lambda i: (0, 0)),
            pl.BlockSpec((1, O), lambda i: (0, 0)),
        ],
        out_specs=[
            pl.BlockSpec((BM, O), lambda i: (i, 0)),
            pl.BlockSpec((2, BM, HALF), lambda i: (0, i, 0)),
        ],
        out_shape=[
            jax.ShapeDtypeStruct((N_NODES, O), jnp.float32),
            jax.ShapeDtypeStruct((2, N_NODES, HALF), jnp.float32),
        ],
    )(x, W_node.T, b_node[None, :], att2, w1t, w3t, b_apply[None, :])

    y2cat3 = pl.pallas_call(
        _edge_fc_body,
        grid=(e_blocks,),
        in_specs=[
            pl.BlockSpec((BE, RD), lambda i: (i, 0)),
            pl.BlockSpec((RD, RD), lambda i: (0, 0)),
            pl.BlockSpec((1, RD), lambda i: (0, 0)),
            pl.BlockSpec((BE, 1), lambda i: (i, 0)),
            pl.BlockSpec((RD, O), lambda i: (0, 0)),
        ],
        out_specs=pl.BlockSpec((2, BE, HALF), lambda i: (0, i, 0)),
        out_shape=jax.ShapeDtypeStruct((2, N_EDGES, HALF), jnp.float32),
    )(edge_feat, W_rel.T, b_rel[None, :], eatt2, w2t)

    src = edge_index[0]
    dst = edge_index[1]
    zeros128 = jnp.zeros((SLAB, HALF), jnp.float32)

    sc_mesh = plsc.VectorSubcoreMesh(core_axis_name="c", subcore_axis_name="s",
                                     num_cores=2, num_subcores=N_SUBCORES)
    agg_u = pl.kernel(
        _sc_u_scatter,
        out_type=jax.ShapeDtypeStruct((2, N_NODES, HALF), jnp.float32),
        mesh=sc_mesh,
        scratch_types=[
            pltpu.VMEM((CHUNK,), jnp.int32),
            pltpu.VMEM((CHUNK,), jnp.int32),
            pltpu.VMEM((CHUNK,), jnp.int32),
            pltpu.VMEM((CHUNK,), jnp.int32),
            pltpu.VMEM((CHUNK, HALF), jnp.float32),
            pltpu.VMEM((CHUNK, HALF), jnp.float32),
            pltpu.SemaphoreType.DMA,
            pltpu.SemaphoreType.DMA,
            pltpu.SemaphoreType.DMA,
            pltpu.SemaphoreType.DMA,
            pltpu.VMEM((SLAB, HALF), jnp.float32),
            pltpu.VMEM_SHARED((N_NODES, HALF), jnp.float32),
        ],
    )(ucat3, src, dst, zeros128)

    agg = pl.kernel(
        _sc_y2_scatter,
        out_type=jax.ShapeDtypeStruct((2, N_NODES, HALF), jnp.float32),
        mesh=sc_mesh,
        scratch_types=[
            pltpu.VMEM((CHUNK,), jnp.int32),
            pltpu.VMEM((CHUNK,), jnp.int32),
            pltpu.VMEM((CHUNK, HALF), jnp.float32),
            pltpu.VMEM((CHUNK, HALF), jnp.float32),
            pltpu.SemaphoreType.DMA,
            pltpu.SemaphoreType.DMA,
            pltpu.SemaphoreType.DMA,
            pltpu.SemaphoreType.DMA,
            pltpu.VMEM((SLAB, HALF), jnp.float32),
            pltpu.VMEM_SHARED((N_NODES, HALF), jnp.float32),
        ],
    )(y2cat3, dst, agg_u)

    out = pl.pallas_call(
        _apply_body,
        grid=(n_blocks,),
        in_specs=[
            pl.BlockSpec((2, BM, HALF), lambda i: (0, i, 0)),
            pl.BlockSpec((BM, O), lambda i: (i, 0)),
            pl.BlockSpec((BM, 1), lambda i: (i, 0)),
        ],
        out_specs=pl.BlockSpec((BM, O), lambda i: (i, 0)),
        out_shape=jax.ShapeDtypeStruct((N_NODES, O), jnp.float32),
    )(agg, v, att2)

    return out


# speedup vs baseline: 5.6960x; 1.1886x over previous
import jax
import jax.numpy as jnp
from jax import lax
from jax.experimental import pallas as pl
from jax.experimental.pallas import tpu as pltpu
from jax.experimental.pallas import tpu_sc as plsc

N_NODES = 10000
N_EDGES = 160000
D = 256
RD = 16
O = 256
HALF = 128

BM = 1000
BEP = 1000
CHUNK = 128
N_CHUNKS = N_EDGES // CHUNK
N_SUBCORES = 16
ROWS_PER_TILE = 624
TAIL_BASE = ROWS_PER_TILE * N_SUBCORES
TAIL_ROWS = N_NODES - TAIL_BASE
SLAB = 48
SLABS_PER_TILE = ROWS_PER_TILE // SLAB


def _node_fc_body(x_ref, wnt_ref, bn_ref, att_ref, w1t_ref, w3t_ref, ba_ref,
                  v_ref, u_ref):
    h = jnp.dot(x_ref[...], wnt_ref[...], preferred_element_type=jnp.float32)
    h = h + bn_ref[...]
    v_ref[...] = jnp.dot(h, w3t_ref[...],
                         preferred_element_type=jnp.float32) + ba_ref[...]
    u = jnp.dot(att_ref[...] * h, w1t_ref[...],
                preferred_element_type=jnp.float32)
    u_ref[0] = u[:, :HALF]
    u_ref[1] = u[:, HALF:]


def _edge_fc_body(efp_ref, wfull_ref, w2a_ref, w2b_ref, ya_ref, yb_ref):
    z2p = jnp.dot(efp_ref[...], wfull_ref[...],
                  preferred_element_type=jnp.float32)
    ya = jnp.dot(z2p, w2a_ref[...], preferred_element_type=jnp.float32)
    yb = jnp.dot(z2p, w2b_ref[...], preferred_element_type=jnp.float32)
    ya_ref[...] = ya.reshape(8 * BEP, HALF)
    yb_ref[...] = yb.reshape(8 * BEP, HALF)


def _apply_body(agg_ref, v_ref, att_ref, out_ref):
    att = att_ref[...]
    v = v_ref[...]
    out_ref[:, :HALF] = att * jnp.maximum(agg_ref[0] + v[:, :HALF], 0.0)
    out_ref[:, HALF:] = att * jnp.maximum(agg_ref[1] + v[:, HALF:], 0.0)


def _sc_u_scatter(ucat_hbm, src_hbm, dst_hbm, zeros_hbm, out_hbm,
                  src0, src1, dst0, dst1, rows0, rows1,
                  isem0, isem1, gsem0, gsem1, buf, accum):
    c = lax.axis_index("c")
    s = lax.axis_index("s")
    r0 = s * ROWS_PER_TILE

    pltpu.sync_copy(zeros_hbm, buf)
    for k in range(SLABS_PER_TILE):
        pltpu.sync_copy(buf, accum.at[pl.ds(r0 + k * SLAB, SLAB)])

    @pl.when(s == N_SUBCORES - 1)
    def _():
        pltpu.sync_copy(buf.at[pl.ds(0, TAIL_ROWS)],
                        accum.at[pl.ds(TAIL_BASE, TAIL_ROWS)])

    plsc.subcore_barrier()

    lo = (s * N_CHUNKS) // N_SUBCORES
    hi = ((s + 1) * N_CHUNKS) // N_SUBCORES

    def issue_idx(j, src_v, dst_v, isem):
        base = j * CHUNK
        pltpu.async_copy(src_hbm.at[pl.ds(base, CHUNK)], src_v, isem)
        pltpu.async_copy(dst_hbm.at[pl.ds(base, CHUNK)], dst_v, isem)

    def drain_idx(src_v, dst_v, isem):
        pltpu.make_async_copy(src_hbm.at[pl.ds(0, CHUNK)], src_v,
                              isem).wait()
        pltpu.make_async_copy(dst_hbm.at[pl.ds(0, CHUNK)], dst_v,
                              isem).wait()

    def issue_gather(src_v, rows_v, gsem):
        pltpu.async_copy(ucat_hbm.at[c].at[src_v], rows_v, gsem)

    def drain_gather(rows_v, gsem):
        pltpu.make_async_copy(ucat_hbm.at[c, pl.ds(0, CHUNK)], rows_v,
                              gsem).wait()

    issue_idx(lo, src0, dst0, isem0)
    drain_idx(src0, dst0, isem0)
    issue_gather(src0, rows0, gsem0)
    issue_idx(lo + 1, src1, dst1, isem1)

    def body(j, carry):
        je = j + 1

        @pl.when(lax.rem(j - lo, 2) == 0)
        def _():
            drain_gather(rows0, gsem0)

            @pl.when(je < hi)
            def _():
                drain_idx(src1, dst1, isem1)
                issue_gather(src1, rows1, gsem1)

            pltpu.sync_copy(rows0, accum.at[dst0], add=True)

            @pl.when(j + 2 < hi)
            def _():
                issue_idx(j + 2, src0, dst0, isem0)

        @pl.when(lax.rem(j - lo, 2) == 1)
        def _():
            drain_gather(rows1, gsem1)

            @pl.when(je < hi)
            def _():
                drain_idx(src0, dst0, isem0)
                issue_gather(src0, rows0, gsem0)

            pltpu.sync_copy(rows1, accum.at[dst1], add=True)

            @pl.when(j + 2 < hi)
            def _():
                issue_idx(j + 2, src1, dst1, isem1)

        return carry

    lax.fori_loop(lo, hi, body, 0)
    plsc.subcore_barrier()

    for k in range(SLABS_PER_TILE):
        rr = r0 + k * SLAB
        pltpu.sync_copy(accum.at[pl.ds(rr, SLAB)], buf)
        pltpu.sync_copy(buf, out_hbm.at[c, pl.ds(rr, SLAB)])

    @pl.when(s == N_SUBCORES - 1)
    def _():
        pltpu.sync_copy(accum.at[pl.ds(TAIL_BASE, TAIL_ROWS)],
                        buf.at[pl.ds(0, TAIL_ROWS)])
        pltpu.sync_copy(buf.at[pl.ds(0, TAIL_ROWS)],
                        out_hbm.at[c, pl.ds(TAIL_BASE, TAIL_ROWS)])


def _sc_y2_scatter(y2a_hbm, y2b_hbm, dst_hbm, aggu_hbm, out_hbm,
                   dst0, dst1, rows0, rows1,
                   isem0, isem1, lsem0, lsem1, buf, accum):
    c = lax.axis_index("c")
    s = lax.axis_index("s")
    r0 = s * ROWS_PER_TILE

    for k in range(SLABS_PER_TILE):
        rr = r0 + k * SLAB
        pltpu.sync_copy(aggu_hbm.at[c, pl.ds(rr, SLAB)], buf)
        pltpu.sync_copy(buf, accum.at[pl.ds(rr, SLAB)])

    @pl.when(s == N_SUBCORES - 1)
    def _():
        pltpu.sync_copy(aggu_hbm.at[c, pl.ds(TAIL_BASE, TAIL_ROWS)],
                        buf.at[pl.ds(0, TAIL_ROWS)])
        pltpu.sync_copy(buf.at[pl.ds(0, TAIL_ROWS)],
                        accum.at[pl.ds(TAIL_BASE, TAIL_ROWS)])

    plsc.subcore_barrier()

    lo = (s * N_CHUNKS) // N_SUBCORES
    hi = ((s + 1) * N_CHUNKS) // N_SUBCORES

    def issue(j, dst_v, rows_v, isem, lsem):
        base = j * CHUNK
        pltpu.async_copy(dst_hbm.at[pl.ds(base, CHUNK)], dst_v, isem)

        @pl.when(c == 0)
        def _():
            pltpu.async_copy(y2a_hbm.at[pl.ds(base, CHUNK)], rows_v, lsem)

        @pl.when(c == 1)
        def _():
            pltpu.async_copy(y2b_hbm.at[pl.ds(base, CHUNK)], rows_v, lsem)

    def drain(dst_v, rows_v, isem, lsem):
        pltpu.make_async_copy(dst_hbm.at[pl.ds(0, CHUNK)], dst_v,
                              isem).wait()
        pltpu.make_async_copy(y2a_hbm.at[pl.ds(0, CHUNK)], rows_v,
                              lsem).wait()

    issue(lo, dst0, rows0, isem0, lsem0)

    def body(j, carry):
        je = j + 1

        @pl.when(lax.rem(j - lo, 2) == 0)
        def _():
            drain(dst0, rows0, isem0, lsem0)

            @pl.when(je < hi)
            def _():
                issue(je, dst1, rows1, isem1, lsem1)

            pltpu.sync_copy(rows0, accum.at[dst0], add=True)

        @pl.when(lax.rem(j - lo, 2) == 1)
        def _():
            drain(dst1, rows1, isem1, lsem1)

            @pl.when(je < hi)
            def _():
                issue(je, dst0, rows0, isem0, lsem0)

            pltpu.sync_copy(rows1, accum.at[dst1], add=True)

        return carry

    lax.fori_loop(lo, hi, body, 0)
    plsc.subcore_barrier()

    for k in range(SLABS_PER_TILE):
        rr = r0 + k * SLAB
        pltpu.sync_copy(accum.at[pl.ds(rr, SLAB)], buf)
        pltpu.sync_copy(buf, out_hbm.at[c, pl.ds(rr, SLAB)])

    @pl.when(s == N_SUBCORES - 1)
    def _():
        pltpu.sync_copy(accum.at[pl.ds(TAIL_BASE, TAIL_ROWS)],
                        buf.at[pl.ds(0, TAIL_ROWS)])
        pltpu.sync_copy(buf.at[pl.ds(0, TAIL_ROWS)],
                        out_hbm.at[c, pl.ds(TAIL_BASE, TAIL_ROWS)])


def kernel(x, node_att, edge_index, edge_feat, edge_att,
           W_node, b_node, W_rel, b_rel, W_apply, b_apply):
    n_blocks = N_NODES // BM

    att2 = node_att[:, None]
    eatt2 = edge_att[:, None]
    w1t = W_apply[:, :D].T
    w2t = W_apply[:, D:D + RD].T
    w3t = W_apply[:, D + RD:].T

    v, ucat3 = pl.pallas_call(
        _node_fc_body,
        grid=(n_blocks,),
        in_specs=[
            pl.BlockSpec((BM, D), lambda i: (i, 0)),
            pl.BlockSpec((D, D), lambda i: (0, 0)),
            pl.BlockSpec((1, D), lambda i: (0, 0)),
            pl.BlockSpec((BM, 1), lambda i: (i, 0)),
            pl.BlockSpec((D, O), lambda i: (0, 0)),
            pl.BlockSpec((D, O), lambda i: (0, 0)),
            pl.BlockSpec((1, O), lambda i: (0, 0)),
        ],
        out_specs=[
            pl.BlockSpec((BM, O), lambda i: (i, 0)),
            pl.BlockSpec((2, BM, HALF), lambda i: (0, i, 0)),
        ],
        out_shape=[
            jax.ShapeDtypeStruct((N_NODES, O), jnp.float32),
            jax.ShapeDtypeStruct((2, N_NODES, HALF), jnp.float32),
        ],
    )(x, W_node.T, b_node[None, :], att2, w1t, w3t, b_apply[None, :])

    efr_scaled = (edge_feat * eatt2).reshape(N_EDGES // 8, 8 * RD)
    ef_pack = jnp.concatenate(
        [efr_scaled, edge_att.reshape(N_EDGES // 8, 8)], axis=1)
    wrb = jax.scipy.linalg.block_diag(*([W_rel.T] * 8))
    bias_rows = jnp.kron(jnp.eye(8, dtype=jnp.float32), b_rel[None, :])
    wfull = jnp.concatenate([wrb, bias_rows], axis=0)
    w2a = jax.scipy.linalg.block_diag(*([w2t[:, :HALF]] * 8))
    w2b = jax.scipy.linalg.block_diag(*([w2t[:, HALF:]] * 8))

    y2a, y2b = pl.pallas_call(
        _edge_fc_body,
        grid=(N_EDGES // (8 * BEP),),
        in_specs=[
            pl.BlockSpec((BEP, 8 * RD + 8), lambda i: (i, 0)),
            pl.BlockSpec((8 * RD + 8, 8 * RD), lambda i: (0, 0)),
            pl.BlockSpec((8 * RD, 8 * HALF), lambda i: (0, 0)),
            pl.BlockSpec((8 * RD, 8 * HALF), lambda i: (0, 0)),
        ],
        out_specs=[
            pl.BlockSpec((8 * BEP, HALF), lambda i: (i, 0)),
            pl.BlockSpec((8 * BEP, HALF), lambda i: (i, 0)),
        ],
        out_shape=[
            jax.ShapeDtypeStruct((N_EDGES, HALF), jnp.float32),
            jax.ShapeDtypeStruct((N_EDGES, HALF), jnp.float32),
        ],
    )(ef_pack, wfull, w2a, w2b)

    src = edge_index[0]
    dst = edge_index[1]
    zeros128 = jnp.zeros((SLAB, HALF), jnp.float32)

    sc_mesh = plsc.VectorSubcoreMesh(core_axis_name="c", subcore_axis_name="s",
                                     num_cores=2, num_subcores=N_SUBCORES)
    agg_u = pl.kernel(
        _sc_u_scatter,
        out_type=jax.ShapeDtypeStruct((2, N_NODES, HALF), jnp.float32),
        mesh=sc_mesh,
        scratch_types=[
            pltpu.VMEM((CHUNK,), jnp.int32),
            pltpu.VMEM((CHUNK,), jnp.int32),
            pltpu.VMEM((CHUNK,), jnp.int32),
            pltpu.VMEM((CHUNK,), jnp.int32),
            pltpu.VMEM((CHUNK, HALF), jnp.float32),
            pltpu.VMEM((CHUNK, HALF), jnp.float32),
            pltpu.SemaphoreType.DMA,
            pltpu.SemaphoreType.DMA,
            pltpu.SemaphoreType.DMA,
            pltpu.SemaphoreType.DMA,
            pltpu.VMEM((SLAB, HALF), jnp.float32),
            pltpu.VMEM_SHARED((N_NODES, HALF), jnp.float32),
        ],
    )(ucat3, src, dst, zeros128)

    agg = pl.kernel(
        _sc_y2_scatter,
        out_type=jax.ShapeDtypeStruct((2, N_NODES, HALF), jnp.float32),
        mesh=sc_mesh,
        scratch_types=[
            pltpu.VMEM((CHUNK,), jnp.int32),
            pltpu.VMEM((CHUNK,), jnp.int32),
            pltpu.VMEM((CHUNK, HALF), jnp.float32),
            pltpu.VMEM((CHUNK, HALF), jnp.float32),
            pltpu.SemaphoreType.DMA,
            pltpu.SemaphoreType.DMA,
            pltpu.SemaphoreType.DMA,
            pltpu.SemaphoreType.DMA,
            pltpu.VMEM((SLAB, HALF), jnp.float32),
            pltpu.VMEM_SHARED((N_NODES, HALF), jnp.float32),
        ],
    )(y2a, y2b, dst, agg_u)

    out = pl.pallas_call(
        _apply_body,
        grid=(n_blocks,),
        in_specs=[
            pl.BlockSpec((2, BM, HALF), lambda i: (0, i, 0)),
            pl.BlockSpec((BM, O), lambda i: (i, 0)),
            pl.BlockSpec((BM, 1), lambda i: (i, 0)),
        ],
        out_specs=pl.BlockSpec((BM, O), lambda i: (i, 0)),
        out_shape=jax.ShapeDtypeStruct((N_NODES, O), jnp.float32),
    )(agg, v, att2)

    return out
